# Initial kernel scaffold; baseline (speedup 1.0000x reference)
import jax
import jax.numpy as jnp
from jax.experimental import pallas as pl


def kernel(adj_indices, adj_values, embedding):
    raise NotImplementedError("write your pallas kernel here")


# speedup vs baseline: 5.8411x; 5.8411x over previous
import functools

import jax
import jax.numpy as jnp
from jax import lax
from jax.experimental import pallas as pl
from jax.experimental.pallas import tpu as pltpu
from jax.experimental.pallas import tpu_sc as plsc

N = 10000
D = 128
E = 320000
LAYERS = 3
NC, NS = 2, 16
NW = NC * NS
E_TILE = E // NW
BLK = 80
NBLK = E_TILE // BLK
NCH = 5
CB = NBLK // NCH
ACC_ROWS = 10240
ROWS_PER_TILE = ACC_ROWS // NS
ZCH = 64


def _spmm_body(src_hbm, dst_hbm, vals_hbm, emb_hbm, out_hbm,
               src_v, dst_v, vals_v, rows_v, stage_v, gsem, acc_sh):
    c = lax.axis_index("c")
    s = lax.axis_index("s")
    wid = s * NC + c

    zero = jnp.zeros((16,), jnp.float32)

    def zrow(i, carry):
        for q in range(D // 16):
            stage_v[i, pl.ds(q * 16, 16)] = zero
        return carry

    lax.fori_loop(0, ZCH, zrow, 0)
    row0 = s * ROWS_PER_TILE
    for t in range(ROWS_PER_TILE // ZCH):
        pltpu.sync_copy(stage_v, acc_sh.at[pl.ds(row0 + t * ZCH, ZCH)])
    plsc.subcore_barrier()

    def chunk(ch, carry):
        pltpu.sync_copy(src_hbm.at[wid, ch], src_v)
        pltpu.sync_copy(dst_hbm.at[wid, ch], dst_v)
        pltpu.sync_copy(vals_hbm.at[wid, ch], vals_v)

        def blk(j, carry2):
            pltpu.async_copy(emb_hbm.at[src_v.at[j]], rows_v, gsem).wait()
            for g in range(BLK // 16):
                vv = vals_v[j, pl.ds(g * 16, 16)]
                for l in range(16):
                    e = g * 16 + l
                    vb = vv[l]
                    for q in range(D // 16):
                        rows_v[e, pl.ds(q * 16, 16)] = (
                            rows_v[e, pl.ds(q * 16, 16)] * vb)
            pltpu.sync_copy(rows_v, acc_sh.at[dst_v.at[j]], add=True)
            return carry2

        lax.fori_loop(0, CB, blk, 0)
        return carry

    lax.fori_loop(0, NCH, chunk, 0)
    plsc.subcore_barrier()

    for t in range(ROWS_PER_TILE // ZCH):
        r = row0 + t * ZCH
        pltpu.sync_copy(acc_sh.at[pl.ds(r, ZCH)], stage_v)
        pltpu.sync_copy(stage_v, out_hbm.at[c, pl.ds(r, ZCH)])


_spmm = functools.partial(
    pl.kernel,
    out_type=jax.ShapeDtypeStruct((NC, ACC_ROWS, D), jnp.float32),
    mesh=plsc.VectorSubcoreMesh(core_axis_name="c", subcore_axis_name="s"),
    scratch_types=[
        pltpu.VMEM((CB, BLK), jnp.int32),
        pltpu.VMEM((CB, BLK), jnp.int32),
        pltpu.VMEM((CB, BLK), jnp.float32),
        pltpu.VMEM((BLK, D), jnp.float32),
        pltpu.VMEM((ZCH, D), jnp.float32),
        pltpu.SemaphoreType.DMA,
        pltpu.VMEM_SHARED((ACC_ROWS, D), jnp.float32),
    ],
)(_spmm_body)


def _combine_body(acc_ref, tot_ref, emb_out, tot_out):
    e = acc_ref[0] + acc_ref[1]
    emb_out[...] = e
    tot_out[...] = tot_ref[...] + e


_RB = 1000

_combine = pl.pallas_call(
    _combine_body,
    grid=(N // _RB,),
    in_specs=[
        pl.BlockSpec((NC, _RB, D), lambda i: (0, i, 0)),
        pl.BlockSpec((_RB, D), lambda i: (i, 0)),
    ],
    out_specs=[
        pl.BlockSpec((_RB, D), lambda i: (i, 0)),
        pl.BlockSpec((_RB, D), lambda i: (i, 0)),
    ],
    out_shape=[jax.ShapeDtypeStruct((N, D), jnp.float32)] * 2,
)


def kernel(adj_indices, adj_values, embedding):
    idx = adj_indices.astype(jnp.int32)
    dst3 = idx[0].reshape(NW, NCH, CB, BLK)
    src3 = idx[1].reshape(NW, NCH, CB, BLK)
    vals3 = adj_values.reshape(NW, NCH, CB, BLK)
    emb = embedding
    total = embedding
    for _ in range(LAYERS):
        acc = _spmm(src3, dst3, vals3, emb)
        emb, total = _combine(acc, total)
    return total


# speedup vs baseline: 8.7376x; 1.4959x over previous
import functools

import jax
import jax.numpy as jnp
from jax import lax
from jax.experimental import pallas as pl
from jax.experimental.pallas import tpu as pltpu
from jax.experimental.pallas import tpu_sc as plsc

N = 10000
D = 128
E = 320000
LAYERS = 3
NC, NS = 2, 16
NW = NC * NS
E_TILE = E // NW
BLK = 80
NBLK = E_TILE // BLK
NCH = 5
CB = NBLK // NCH
ACC_ROWS = 10240
ROWS_PER_TILE = ACC_ROWS // NS
ZCH = 64


def _spmm_body(src_hbm, dst_hbm, vals_hbm, emb_hbm, out_hbm,
               src_v, dst_v, vals_v, rows_v, gsem, ssem, acc_sh):
    c = lax.axis_index("c")
    s = lax.axis_index("s")
    wid = s * NC + c

    zero = jnp.zeros((16,), jnp.float32)

    def zrow(i, carry):
        for q in range(D // 16):
            rows_v[0, i, pl.ds(q * 16, 16)] = zero
        return carry

    lax.fori_loop(0, BLK, zrow, 0)
    row0 = s * ROWS_PER_TILE
    for t in range(ROWS_PER_TILE // BLK):
        pltpu.sync_copy(rows_v.at[0], acc_sh.at[pl.ds(row0 + t * BLK, BLK)])
    plsc.subcore_barrier()

    def chunk(ch, carry):
        pltpu.sync_copy(src_hbm.at[wid, ch], src_v)
        pltpu.sync_copy(dst_hbm.at[wid, ch], dst_v)
        pltpu.sync_copy(vals_hbm.at[wid, ch], vals_v)
        pltpu.async_copy(emb_hbm.at[src_v.at[0]], rows_v.at[0], gsem)

        def blk(j, carry2):
            b = j & 1
            pltpu.make_async_copy(
                emb_hbm.at[src_v.at[j]], rows_v.at[b], gsem).wait()

            @pl.when(j >= 1)
            def _():
                pltpu.make_async_copy(
                    rows_v.at[1 - b], acc_sh.at[dst_v.at[j]], ssem).wait()

            @pl.when(j < CB - 1)
            def _():
                pltpu.async_copy(
                    emb_hbm.at[src_v.at[j + 1]], rows_v.at[1 - b], gsem)

            for g in range(BLK // 16):
                vv = vals_v[j, pl.ds(g * 16, 16)]
                for l in range(16):
                    e = g * 16 + l
                    vb = vv[l]
                    for q in range(D // 16):
                        rows_v[b, e, pl.ds(q * 16, 16)] = (
                            rows_v[b, e, pl.ds(q * 16, 16)] * vb)
            pltpu.async_copy(
                rows_v.at[b], acc_sh.at[dst_v.at[j]], ssem, add=True)
            return carry2

        lax.fori_loop(0, CB, blk, 0)
        pltpu.make_async_copy(
            rows_v.at[(CB - 1) & 1], acc_sh.at[dst_v.at[CB - 1]], ssem).wait()
        return carry

    lax.fori_loop(0, NCH, chunk, 0)
    plsc.subcore_barrier()

    for t in range(ROWS_PER_TILE // BLK):
        r = row0 + t * BLK
        pltpu.sync_copy(acc_sh.at[pl.ds(r, BLK)], rows_v.at[0])
        pltpu.sync_copy(rows_v.at[0], out_hbm.at[c, pl.ds(r, BLK)])


_spmm = functools.partial(
    pl.kernel,
    out_type=jax.ShapeDtypeStruct((NC, ACC_ROWS, D), jnp.float32),
    mesh=plsc.VectorSubcoreMesh(core_axis_name="c", subcore_axis_name="s"),
    scratch_types=[
        pltpu.VMEM((CB, BLK), jnp.int32),
        pltpu.VMEM((CB, BLK), jnp.int32),
        pltpu.VMEM((CB, BLK), jnp.float32),
        pltpu.VMEM((2, BLK, D), jnp.float32),
        pltpu.SemaphoreType.DMA,
        pltpu.SemaphoreType.DMA,
        pltpu.VMEM_SHARED((ACC_ROWS, D), jnp.float32),
    ],
)(_spmm_body)


def _combine_body(acc_ref, tot_ref, emb_out, tot_out):
    e = acc_ref[0] + acc_ref[1]
    emb_out[...] = e
    tot_out[...] = tot_ref[...] + e


_RB = 1000

_combine = pl.pallas_call(
    _combine_body,
    grid=(N // _RB,),
    in_specs=[
        pl.BlockSpec((NC, _RB, D), lambda i: (0, i, 0)),
        pl.BlockSpec((_RB, D), lambda i: (i, 0)),
    ],
    out_specs=[
        pl.BlockSpec((_RB, D), lambda i: (i, 0)),
        pl.BlockSpec((_RB, D), lambda i: (i, 0)),
    ],
    out_shape=[jax.ShapeDtypeStruct((N, D), jnp.float32)] * 2,
)


def kernel(adj_indices, adj_values, embedding):
    idx = adj_indices.astype(jnp.int32)
    dst3 = idx[0].reshape(NW, NCH, CB, BLK)
    src3 = idx[1].reshape(NW, NCH, CB, BLK)
    vals3 = adj_values.reshape(NW, NCH, CB, BLK)
    emb = embedding
    total = embedding
    for _ in range(LAYERS):
        acc = _spmm(src3, dst3, vals3, emb)
        emb, total = _combine(acc, total)
    return total


# speedup vs baseline: 9.5060x; 1.0879x over previous
import functools

import jax
import jax.numpy as jnp
from jax import lax
from jax.experimental import pallas as pl
from jax.experimental.pallas import tpu as pltpu
from jax.experimental.pallas import tpu_sc as plsc

N = 10000
D = 128
E = 320000
LAYERS = 3
NC, NS = 2, 16
NW = NC * NS
E_TILE = E // NW
BLK = 80
NBLK = E_TILE // BLK
NCH = 5
CB = NBLK // NCH
ACC_ROWS = 10240
ROWS_PER_TILE = ACC_ROWS // NS
ZCH = 64


def _spmm_body(src_hbm, dst_hbm, vals_hbm, emb_hbm, out_hbm,
               src_v, dst_v, vals_v, rows_v, gsem, ssem, acc_sh):
    c = lax.axis_index("c")
    s = lax.axis_index("s")
    wid = s * NC + c

    zero = jnp.zeros((16,), jnp.float32)

    def zrow(i, carry):
        for q in range(D // 16):
            rows_v[0, i, pl.ds(q * 16, 16)] = zero
        return carry

    lax.fori_loop(0, BLK, zrow, 0)
    row0 = s * ROWS_PER_TILE
    for t in range(ROWS_PER_TILE // BLK):
        pltpu.sync_copy(rows_v.at[0], acc_sh.at[pl.ds(row0 + t * BLK, BLK)])
    plsc.subcore_barrier()

    def chunk(ch, carry):
        pltpu.sync_copy(src_hbm.at[wid, ch], src_v)
        pltpu.sync_copy(dst_hbm.at[wid, ch], dst_v)
        pltpu.sync_copy(vals_hbm.at[wid, ch], vals_v)
        pltpu.async_copy(emb_hbm.at[src_v.at[0]], rows_v.at[0], gsem)

        def blk(j, carry2):
            b = j & 1

            @pl.when(j >= 1)
            def _():
                pltpu.make_async_copy(
                    rows_v.at[1 - b], acc_sh.at[dst_v.at[j]], ssem).wait()

            @pl.when(j < CB - 1)
            def _():
                pltpu.async_copy(
                    emb_hbm.at[src_v.at[j + 1]], rows_v.at[1 - b], gsem)

            pltpu.make_async_copy(
                emb_hbm.at[src_v.at[j]], rows_v.at[b], gsem).wait()

            for g in range(BLK // 16):
                vv = vals_v[j, pl.ds(g * 16, 16)]
                for l in range(16):
                    e = g * 16 + l
                    vb = vv[l]
                    for q in range(D // 16):
                        rows_v[b, e, pl.ds(q * 16, 16)] = (
                            rows_v[b, e, pl.ds(q * 16, 16)] * vb)
            pltpu.async_copy(
                rows_v.at[b], acc_sh.at[dst_v.at[j]], ssem, add=True)
            return carry2

        lax.fori_loop(0, CB, blk, 0)
        pltpu.make_async_copy(
            rows_v.at[(CB - 1) & 1], acc_sh.at[dst_v.at[CB - 1]], ssem).wait()
        return carry

    lax.fori_loop(0, NCH, chunk, 0)
    plsc.subcore_barrier()

    for t in range(ROWS_PER_TILE // BLK):
        r = row0 + t * BLK
        pltpu.sync_copy(acc_sh.at[pl.ds(r, BLK)], rows_v.at[0])
        pltpu.sync_copy(rows_v.at[0], out_hbm.at[c, pl.ds(r, BLK)])


_spmm = functools.partial(
    pl.kernel,
    out_type=jax.ShapeDtypeStruct((NC, ACC_ROWS, D), jnp.float32),
    mesh=plsc.VectorSubcoreMesh(core_axis_name="c", subcore_axis_name="s"),
    scratch_types=[
        pltpu.VMEM((CB, BLK), jnp.int32),
        pltpu.VMEM((CB, BLK), jnp.int32),
        pltpu.VMEM((CB, BLK), jnp.float32),
        pltpu.VMEM((2, BLK, D), jnp.float32),
        pltpu.SemaphoreType.DMA,
        pltpu.SemaphoreType.DMA,
        pltpu.VMEM_SHARED((ACC_ROWS, D), jnp.float32),
    ],
)(_spmm_body)


def _combine_body(acc_ref, tot_ref, emb_out, tot_out):
    e = acc_ref[0] + acc_ref[1]
    emb_out[...] = e
    tot_out[...] = tot_ref[...] + e


_RB = 1000

_combine = pl.pallas_call(
    _combine_body,
    grid=(N // _RB,),
    in_specs=[
        pl.BlockSpec((NC, _RB, D), lambda i: (0, i, 0)),
        pl.BlockSpec((_RB, D), lambda i: (i, 0)),
    ],
    out_specs=[
        pl.BlockSpec((_RB, D), lambda i: (i, 0)),
        pl.BlockSpec((_RB, D), lambda i: (i, 0)),
    ],
    out_shape=[jax.ShapeDtypeStruct((N, D), jnp.float32)] * 2,
)


def kernel(adj_indices, adj_values, embedding):
    idx = adj_indices.astype(jnp.int32)
    dst3 = idx[0].reshape(NW, NCH, CB, BLK)
    src3 = idx[1].reshape(NW, NCH, CB, BLK)
    vals3 = adj_values.reshape(NW, NCH, CB, BLK)
    emb = embedding
    total = embedding
    for _ in range(LAYERS):
        acc = _spmm(src3, dst3, vals3, emb)
        emb, total = _combine(acc, total)
    return total


# speedup vs baseline: 9.6724x; 1.0175x over previous
import functools

import jax
import jax.numpy as jnp
from jax import lax
from jax.experimental import pallas as pl
from jax.experimental.pallas import tpu as pltpu
from jax.experimental.pallas import tpu_sc as plsc

N = 10000
D = 128
E = 320000
LAYERS = 3
NC, NS = 2, 16
NW = NC * NS
E_TILE = E // NW
BLK = 80
NBLK = E_TILE // BLK
NCH = 5
CB = NBLK // NCH
ACC_ROWS = 10240
ROWS_PER_TILE = ACC_ROWS // NS


def _spmm_body(eds_hbm, vals_hbm, emb_hbm, out_hbm, eds_v, vals_v, rows_v, gsem, ssem, acc_sh):
    c = lax.axis_index("c")
    s = lax.axis_index("s")
    wid = s * NC + c

    zero = jnp.zeros((16,), jnp.float32)

    def zrow(i, carry):
        for q in range(D // 16):
            rows_v[0, i, pl.ds(q * 16, 16)] = zero
        return carry

    lax.fori_loop(0, BLK, zrow, 0)
    row0 = s * ROWS_PER_TILE
    for t in range(ROWS_PER_TILE // BLK):
        pltpu.async_copy(
            rows_v.at[0], acc_sh.at[pl.ds(row0 + t * BLK, BLK)], ssem)
    for t in range(ROWS_PER_TILE // BLK):
        pltpu.make_async_copy(
            rows_v.at[0], acc_sh.at[pl.ds(row0, BLK)], ssem).wait()
    plsc.subcore_barrier()

    def chunk(ch, carry):
        pltpu.sync_copy(eds_hbm.at[wid, ch], eds_v)
        pltpu.sync_copy(vals_hbm.at[wid, ch], vals_v)
        pltpu.async_copy(emb_hbm.at[eds_v.at[0]], rows_v.at[0], gsem)

        def blk(j, carry2):
            b = j & 1

            @pl.when(j >= 1)
            def _():
                pltpu.make_async_copy(
                    rows_v.at[1 - b], acc_sh.at[eds_v.at[CB + j]], ssem).wait()

            @pl.when(j < CB - 1)
            def _():
                pltpu.async_copy(
                    emb_hbm.at[eds_v.at[j + 1]], rows_v.at[1 - b], gsem)

            pltpu.make_async_copy(
                emb_hbm.at[eds_v.at[j]], rows_v.at[b], gsem).wait()

            for g in range(BLK // 16):
                vv = vals_v[j, pl.ds(g * 16, 16)]
                for l in range(16):
                    e = g * 16 + l
                    vb = vv[l]
                    for q in range(D // 16):
                        rows_v[b, e, pl.ds(q * 16, 16)] = (
                            rows_v[b, e, pl.ds(q * 16, 16)] * vb)
            pltpu.async_copy(
                rows_v.at[b], acc_sh.at[eds_v.at[CB + j]], ssem, add=True)
            return carry2

        lax.fori_loop(0, CB, blk, 0)
        pltpu.make_async_copy(
            rows_v.at[(CB - 1) & 1], acc_sh.at[eds_v.at[2 * CB - 1]], ssem).wait()
        return carry

    lax.fori_loop(0, NCH, chunk, 0)
    plsc.subcore_barrier()

    pltpu.sync_copy(acc_sh.at[pl.ds(row0, ROWS_PER_TILE)],
                    out_hbm.at[c, pl.ds(row0, ROWS_PER_TILE)])


_spmm = functools.partial(
    pl.kernel,
    out_type=jax.ShapeDtypeStruct((NC, ACC_ROWS, D), jnp.float32),
    mesh=plsc.VectorSubcoreMesh(core_axis_name="c", subcore_axis_name="s"),
    scratch_types=[
        pltpu.VMEM((2 * CB, BLK), jnp.int32),
        pltpu.VMEM((CB, BLK), jnp.float32),
        pltpu.VMEM((2, BLK, D), jnp.float32),
        pltpu.SemaphoreType.DMA,
        pltpu.SemaphoreType.DMA,
        pltpu.VMEM_SHARED((ACC_ROWS, D), jnp.float32),
    ],
)(_spmm_body)


def _combine_body(acc_ref, tot_ref, emb_out, tot_out):
    e = acc_ref[0] + acc_ref[1]
    emb_out[...] = e
    tot_out[...] = tot_ref[...] + e


_RB = 1000

_combine = pl.pallas_call(
    _combine_body,
    grid=(N // _RB,),
    in_specs=[
        pl.BlockSpec((NC, _RB, D), lambda i: (0, i, 0)),
        pl.BlockSpec((_RB, D), lambda i: (i, 0)),
    ],
    out_specs=[
        pl.BlockSpec((_RB, D), lambda i: (i, 0)),
        pl.BlockSpec((_RB, D), lambda i: (i, 0)),
    ],
    out_shape=[jax.ShapeDtypeStruct((N, D), jnp.float32)] * 2,
)


def kernel(adj_indices, adj_values, embedding):
    idx = adj_indices.astype(jnp.int32)
    eds = jnp.concatenate(
        [idx[1].reshape(NW, NCH, CB, BLK),
         idx[0].reshape(NW, NCH, CB, BLK)], axis=2)
    vals = adj_values.reshape(NW, NCH, CB, BLK)
    emb = embedding
    total = embedding
    for _ in range(LAYERS):
        acc = _spmm(eds, vals, emb)
        emb, total = _combine(acc, total)
    return total
